# Initial kernel scaffold; baseline (speedup 1.0000x reference)
import jax
import jax.numpy as jnp
from jax.experimental import pallas as pl


def kernel(word, tag, pos1, pos2, word_W, tag_W, pos1_W, pos2_W):
    raise NotImplementedError("write your pallas kernel here")


# speedup vs baseline: 4.3763x; 4.3763x over previous
import functools

import jax
import jax.numpy as jnp
from jax import lax
from jax.experimental import pallas as pl
from jax.experimental.pallas import tpu as pltpu
from jax.experimental.pallas import tpu_sc as plsc

WORD_DIM = 128
SMALL_DIM = 16
OUT_DIM = WORD_DIM + 3 * SMALL_DIM
CHUNK = 128


@functools.partial(jax.jit, static_argnames=("n_tokens",))
def _embed_2d(wi, ti, p1i, p2i, word_W, tag_W, pos1_W, pos2_W, *, n_tokens):
    info = plsc.get_sparse_core_info()
    nc, ns = info.num_cores, info.num_subcores
    nw = nc * ns
    tpw = n_tokens // nw
    assert tpw * nw == n_tokens and tpw % CHUNK == 0
    steps = tpw // CHUNK

    mesh = plsc.VectorSubcoreMesh(core_axis_name="c", subcore_axis_name="s")

    @functools.partial(
        pl.kernel,
        mesh=mesh,
        compiler_params=pltpu.CompilerParams(use_tc_tiling_on_sc=False),
        out_type=jax.ShapeDtypeStruct((n_tokens, OUT_DIM), jnp.float32),
        scratch_types=[
            pltpu.VMEM((CHUNK,), jnp.int32),
            pltpu.VMEM((CHUNK,), jnp.int32),
            pltpu.VMEM((CHUNK,), jnp.int32),
            pltpu.VMEM((CHUNK,), jnp.int32),
            pltpu.VMEM((CHUNK, WORD_DIM), jnp.float32),
            pltpu.VMEM((CHUNK, SMALL_DIM), jnp.float32),
            pltpu.VMEM((CHUNK, SMALL_DIM), jnp.float32),
            pltpu.VMEM((CHUNK, SMALL_DIM), jnp.float32),
            pltpu.SemaphoreType.DMA,
            pltpu.SemaphoreType.DMA,
            pltpu.SemaphoreType.DMA,
            pltpu.SemaphoreType.DMA,
        ],
    )
    def embed_kernel(wi_h, ti_h, p1_h, p2_h, wW_h, tW_h, p1W_h, p2W_h,
                     out_h, iw_v, it_v, i1_v, i2_v, wbuf, tbuf, b1, b2,
                     sw, st, s1, s2):
        wid = lax.axis_index("s") * nc + lax.axis_index("c")
        w_base = wid * tpw

        def step(i, _):
            base = w_base + i * CHUNK
            tok = pl.ds(base, CHUNK)
            pltpu.sync_copy(wi_h.at[tok], iw_v)
            pltpu.sync_copy(ti_h.at[tok], it_v)
            pltpu.sync_copy(p1_h.at[tok], i1_v)
            pltpu.sync_copy(p2_h.at[tok], i2_v)
            cw = pltpu.async_copy(wW_h.at[iw_v], wbuf, sw)
            ct = pltpu.async_copy(tW_h.at[it_v], tbuf, st)
            c1 = pltpu.async_copy(p1W_h.at[i1_v], b1, s1)
            c2 = pltpu.async_copy(p2W_h.at[i2_v], b2, s2)
            cw.wait()
            ct.wait()
            c1.wait()
            c2.wait()
            pltpu.sync_copy(wbuf, out_h.at[tok, pl.ds(0, WORD_DIM)])
            pltpu.sync_copy(tbuf, out_h.at[tok, pl.ds(WORD_DIM, SMALL_DIM)])
            pltpu.sync_copy(b1, out_h.at[tok, pl.ds(WORD_DIM + SMALL_DIM, SMALL_DIM)])
            pltpu.sync_copy(b2, out_h.at[tok, pl.ds(WORD_DIM + 2 * SMALL_DIM, SMALL_DIM)])
            return 0

        lax.fori_loop(0, steps, step, 0)

    return embed_kernel(wi, ti, p1i, p2i, word_W, tag_W, pos1_W, pos2_W)


def kernel(word, tag, pos1, pos2, word_W, tag_W, pos1_W, pos2_W):
    B, L = word.shape
    n = B * L
    out = _embed_2d(
        word.reshape(n).astype(jnp.int32),
        tag.reshape(n).astype(jnp.int32),
        pos1.reshape(n).astype(jnp.int32),
        pos2.reshape(n).astype(jnp.int32),
        word_W, tag_W, pos1_W, pos2_W,
        n_tokens=n,
    )
    return out.reshape(B, L, OUT_DIM)


# speedup vs baseline: 4.4439x; 1.0154x over previous
import functools

import jax
import jax.numpy as jnp
from jax import lax
from jax.experimental import pallas as pl
from jax.experimental.pallas import tpu as pltpu
from jax.experimental.pallas import tpu_sc as plsc

WORD_DIM = 128
SMALL_DIM = 16
OUT_DIM = WORD_DIM + 3 * SMALL_DIM
IDXW = 128
RPC = 2
CHUNK = IDXW * RPC


@functools.partial(jax.jit, static_argnames=("n_tokens",))
def _embed_2d(wi, ti, p1i, p2i, word_W, tag_W, pos1_W, pos2_W, *, n_tokens):
    info = plsc.get_sparse_core_info()
    nc, ns = info.num_cores, info.num_subcores
    nw = nc * ns
    tpw = n_tokens // nw
    steps = tpw // CHUNK
    assert tpw * nw == n_tokens and steps * CHUNK == tpw and steps % 2 == 0
    half = steps // 2
    rows_pw = tpw // IDXW

    mesh = plsc.VectorSubcoreMesh(core_axis_name="c", subcore_axis_name="s")

    idx_buf = pltpu.VMEM((RPC, IDXW), jnp.int32)
    data_bufs = [
        pltpu.VMEM((CHUNK, WORD_DIM), jnp.float32),
        pltpu.VMEM((CHUNK, SMALL_DIM), jnp.float32),
        pltpu.VMEM((CHUNK, SMALL_DIM), jnp.float32),
        pltpu.VMEM((CHUNK, SMALL_DIM), jnp.float32),
    ]
    per_set = [idx_buf] * 4 + data_bufs + [pltpu.SemaphoreType.DMA] * 3

    @functools.partial(
        pl.kernel,
        mesh=mesh,
        compiler_params=pltpu.CompilerParams(use_tc_tiling_on_sc=False),
        out_type=jax.ShapeDtypeStruct((n_tokens, OUT_DIM), jnp.float32),
        scratch_types=per_set + per_set,
    )
    def embed_kernel(wi_h, ti_h, p1_h, p2_h, wW_h, tW_h, p1W_h, p2W_h,
                     out_h, *scratch):
        wid = lax.axis_index("s") * nc + lax.axis_index("c")

        n_per = len(per_set)
        sets = []
        for b in (0, 1):
            sc = scratch[b * n_per:(b + 1) * n_per]
            sets.append(dict(
                idx=sc[0:4], data=sc[4:8], isem=sc[8], gsem=sc[9], wsem=sc[10],
            ))
        idx_srcs = (wi_h, ti_h, p1_h, p2_h)
        tables = (wW_h, tW_h, p1W_h, p2W_h)
        col_off = (0, WORD_DIM, WORD_DIM + SMALL_DIM, WORD_DIM + 2 * SMALL_DIM)
        col_w = (WORD_DIM, SMALL_DIM, SMALL_DIM, SMALL_DIM)

        def idx_copies(b, i, mk):
            s = sets[b]
            r = wid * rows_pw + i * RPC
            return [mk(idx_srcs[t].at[pl.ds(r, RPC)], s["idx"][t], s["isem"])
                    for t in range(4)]

        def gather_copies(b, mk):
            s = sets[b]
            out = []
            for t in range(4):
                for j in range(RPC):
                    out.append(mk(tables[t].at[s["idx"][t].at[j]],
                                  s["data"][t].at[pl.ds(j * IDXW, IDXW)],
                                  s["gsem"]))
            return out

        def write_copies(b, i, mk):
            s = sets[b]
            tok = pl.ds(wid * tpw + i * CHUNK, CHUNK)
            return [mk(s["data"][t], out_h.at[tok, pl.ds(col_off[t], col_w[t])],
                       s["wsem"])
                    for t in range(4)]

        fire = pltpu.async_copy

        def drain(copier, *args):
            for c in copier(*args, pltpu.make_async_copy):
                c.wait()

        idx_copies(0, 0, fire)
        drain(idx_copies, 0, 0)
        gather_copies(0, fire)
        idx_copies(1, 1, fire)
        drain(idx_copies, 1, 1)
        gather_copies(1, fire)
        drain(gather_copies, 0)
        write_copies(0, 0, fire)
        idx_copies(0, 2, fire)

        def pair(h, _):
            e = 2 * h
            o = e + 1
            drain(idx_copies, 0, e)
            drain(write_copies, 0, e - 2)
            gather_copies(0, fire)
            drain(gather_copies, 1)
            write_copies(1, o - 2, fire)
            idx_copies(1, o, fire)
            drain(idx_copies, 1, o)
            drain(write_copies, 1, o - 2)
            gather_copies(1, fire)
            drain(gather_copies, 0)
            write_copies(0, e, fire)

            @pl.when(h < half - 1)
            def _():
                idx_copies(0, o + 1, fire)

            return 0

        lax.fori_loop(1, half, pair, 0)

        drain(gather_copies, 1)
        write_copies(1, steps - 1, fire)
        drain(write_copies, 0, steps - 2)
        drain(write_copies, 1, steps - 1)

    return embed_kernel(wi, ti, p1i, p2i, word_W, tag_W, pos1_W, pos2_W)


def kernel(word, tag, pos1, pos2, word_W, tag_W, pos1_W, pos2_W):
    B, L = word.shape
    n = B * L
    out = _embed_2d(
        word.reshape(n // IDXW, IDXW).astype(jnp.int32),
        tag.reshape(n // IDXW, IDXW).astype(jnp.int32),
        pos1.reshape(n // IDXW, IDXW).astype(jnp.int32),
        pos2.reshape(n // IDXW, IDXW).astype(jnp.int32),
        word_W, tag_W, pos1_W, pos2_W,
        n_tokens=n,
    )
    return out.reshape(B, L, OUT_DIM)


# speedup vs baseline: 4.4498x; 1.0013x over previous
import functools

import jax
import jax.numpy as jnp
from jax import lax
from jax.experimental import pallas as pl
from jax.experimental.pallas import tpu as pltpu
from jax.experimental.pallas import tpu_sc as plsc

WORD_DIM = 128
SMALL_DIM = 16
OUT_DIM = WORD_DIM + 3 * SMALL_DIM
L_ROW = 200
IDX_SPLIT = (128, 72)


@functools.partial(jax.jit, static_argnames=("batch",))
def _embed(wi, ti, p1i, p2i, word_W, tag_W, pos1_W, pos2_W, *, batch):
    info = plsc.get_sparse_core_info()
    nc, ns = info.num_cores, info.num_subcores
    nw = nc * ns
    bpw = batch // nw
    assert bpw * nw == batch and bpw % 2 == 0
    half = bpw // 2

    mesh = plsc.VectorSubcoreMesh(core_axis_name="c", subcore_axis_name="s")

    per_set = (
        [pltpu.VMEM((L_ROW,), jnp.int32)] * 4
        + [
            pltpu.VMEM((L_ROW, WORD_DIM), jnp.float32),
            pltpu.VMEM((L_ROW, SMALL_DIM), jnp.float32),
            pltpu.VMEM((L_ROW, SMALL_DIM), jnp.float32),
            pltpu.VMEM((L_ROW, SMALL_DIM), jnp.float32),
        ]
        + [pltpu.SemaphoreType.DMA] * 3
    )

    @functools.partial(
        pl.kernel,
        mesh=mesh,
        compiler_params=pltpu.CompilerParams(use_tc_tiling_on_sc=False),
        out_type=jax.ShapeDtypeStruct((batch, L_ROW, OUT_DIM), jnp.float32),
        scratch_types=per_set + per_set,
    )
    def embed_kernel(wi_h, ti_h, p1_h, p2_h, wW_h, tW_h, p1W_h, p2W_h,
                     out_h, *scratch):
        wid = lax.axis_index("s") * nc + lax.axis_index("c")

        n_per = len(per_set)
        sets = []
        for b in (0, 1):
            sc = scratch[b * n_per:(b + 1) * n_per]
            sets.append(dict(
                idx=sc[0:4], data=sc[4:8], isem=sc[8], gsem=sc[9], wsem=sc[10],
            ))
        idx_srcs = (wi_h, ti_h, p1_h, p2_h)
        tables = (wW_h, tW_h, p1W_h, p2W_h)
        col_off = (0, WORD_DIM, WORD_DIM + SMALL_DIM, WORD_DIM + 2 * SMALL_DIM)
        col_w = (WORD_DIM, SMALL_DIM, SMALL_DIM, SMALL_DIM)

        def idx_copies(s, i, mk):
            st = sets[s]
            base = (wid * bpw + i) * L_ROW
            return [mk(idx_srcs[t].at[pl.ds(base, L_ROW)], st["idx"][t],
                       st["isem"])
                    for t in range(4)]

        def gather_copies(s, mk):
            st = sets[s]
            out = []
            for t in range(4):
                off = 0
                for ln in IDX_SPLIT:
                    out.append(mk(tables[t].at[st["idx"][t].at[pl.ds(off, ln)]],
                                  st["data"][t].at[pl.ds(off, ln)],
                                  st["gsem"]))
                    off += ln
            return out

        def write_copies(s, i, mk):
            st = sets[s]
            row = wid * bpw + i
            return [mk(st["data"][t],
                       out_h.at[row, :, pl.ds(col_off[t], col_w[t])],
                       st["wsem"])
                    for t in range(4)]

        fire = pltpu.async_copy

        def drain(copier, *args):
            for c in copier(*args, pltpu.make_async_copy):
                c.wait()

        idx_copies(0, 0, fire)
        drain(idx_copies, 0, 0)
        gather_copies(0, fire)
        idx_copies(1, 1, fire)
        drain(idx_copies, 1, 1)
        gather_copies(1, fire)
        drain(gather_copies, 0)
        write_copies(0, 0, fire)
        idx_copies(0, 2, fire)

        def pair(h, _):
            e = 2 * h
            o = e + 1
            drain(idx_copies, 0, e)
            drain(write_copies, 0, e - 2)
            gather_copies(0, fire)
            drain(gather_copies, 1)
            write_copies(1, o - 2, fire)
            idx_copies(1, o, fire)
            drain(idx_copies, 1, o)
            drain(write_copies, 1, o - 2)
            gather_copies(1, fire)
            drain(gather_copies, 0)
            write_copies(0, e, fire)

            @pl.when(h < half - 1)
            def _():
                idx_copies(0, o + 1, fire)

            return 0

        lax.fori_loop(1, half, pair, 0)

        drain(gather_copies, 1)
        write_copies(1, bpw - 1, fire)
        drain(write_copies, 0, bpw - 2)
        drain(write_copies, 1, bpw - 1)

    return embed_kernel(wi, ti, p1i, p2i, word_W, tag_W, pos1_W, pos2_W)


def kernel(word, tag, pos1, pos2, word_W, tag_W, pos1_W, pos2_W):
    B, L = word.shape
    n = B * L
    return _embed(
        word.reshape(n).astype(jnp.int32),
        tag.reshape(n).astype(jnp.int32),
        pos1.reshape(n).astype(jnp.int32),
        pos2.reshape(n).astype(jnp.int32),
        word_W, tag_W, pos1_W, pos2_W,
        batch=B,
    )
